# baseline (speedup 1.0000x reference)
import functools

import jax
import jax.numpy as jnp
from jax import lax
from jax.experimental import pallas as pl
from jax.experimental.pallas import tpu as pltpu
from jax.experimental.pallas import tpu_sc as plsc

NUM_ACTIONS = 100000
EMBED_DIM = 32
HIDDEN_DIM = 64
LATENT_DIM = 64
BATCH = 16384

NC = 2
NS = 16
NW = NC * NS
B_PER_W = BATCH // NW
CHUNK = 128
NCHUNK = B_PER_W // CHUNK

_sc_mesh = plsc.VectorSubcoreMesh(core_axis_name="c", subcore_axis_name="s")


@functools.partial(
    pl.kernel,
    mesh=_sc_mesh,
    out_type=jax.ShapeDtypeStruct((BATCH, EMBED_DIM), jnp.float32),
    scratch_types=[
        pltpu.VMEM((NCHUNK, CHUNK), jnp.int32),
        pltpu.VMEM((B_PER_W, EMBED_DIM), jnp.float32),
        pltpu.SemaphoreType.DMA,
    ],
    compiler_params=pltpu.CompilerParams(use_tc_tiling_on_sc=False),
)
def _sc_gather(idx_hbm, table_hbm, out_hbm, idx_v, rows_v, sem):
    wid = lax.axis_index("s") * NC + lax.axis_index("c")
    base = wid * B_PER_W
    pltpu.sync_copy(idx_hbm.at[wid], idx_v)
    copies = []
    for j in range(NCHUNK):
        copies.append(
            pltpu.async_copy(
                table_hbm.at[idx_v.at[j]],
                rows_v.at[pl.ds(j * CHUNK, CHUNK)],
                sem,
            )
        )
    for c in copies:
        c.wait()
    pltpu.sync_copy(rows_v, out_hbm.at[pl.ds(base, B_PER_W)])


BLK = 2048
GRID = BATCH // BLK


def _mlp_body(e_ref, w1_ref, b1_ref, gamma_ref, beta_ref, w2_ref, b2_ref,
              out_ref):
    e = e_ref[...]
    h = jnp.dot(e, w1_ref[...], preferred_element_type=jnp.float32)
    h = h + b1_ref[...]
    mean = jnp.mean(h, axis=-1, keepdims=True)
    cen = h - mean
    var = jnp.mean(cen * cen, axis=-1, keepdims=True)
    h = cen * lax.rsqrt(var + 1e-5) * gamma_ref[...] + beta_ref[...]
    h = jnp.maximum(h, 0.0)
    z = jnp.dot(h, w2_ref[...], preferred_element_type=jnp.float32)
    out_ref[...] = z + b2_ref[...]


_mlp = pl.pallas_call(
    _mlp_body,
    grid=(GRID,),
    in_specs=[
        pl.BlockSpec((BLK, EMBED_DIM), lambda i: (i, 0)),
        pl.BlockSpec((EMBED_DIM, HIDDEN_DIM), lambda i: (0, 0)),
        pl.BlockSpec((1, HIDDEN_DIM), lambda i: (0, 0)),
        pl.BlockSpec((1, HIDDEN_DIM), lambda i: (0, 0)),
        pl.BlockSpec((1, HIDDEN_DIM), lambda i: (0, 0)),
        pl.BlockSpec((HIDDEN_DIM, LATENT_DIM), lambda i: (0, 0)),
        pl.BlockSpec((1, LATENT_DIM), lambda i: (0, 0)),
    ],
    out_specs=pl.BlockSpec((BLK, LATENT_DIM), lambda i: (i, 0)),
    out_shape=jax.ShapeDtypeStruct((BATCH, LATENT_DIM), jnp.float32),
)


def kernel(x, table, W1, b1, gamma, beta, W2, b2):
    idx = x.astype(jnp.int32).reshape(NW, NCHUNK, CHUNK)
    e = _sc_gather(idx, table)
    return _mlp(
        e,
        W1,
        b1.reshape(1, HIDDEN_DIM),
        gamma.reshape(1, HIDDEN_DIM),
        beta.reshape(1, HIDDEN_DIM),
        W2,
        b2.reshape(1, LATENT_DIM),
    )
